# baseline (device time: 59600 ns/iter reference)
import jax
import jax.numpy as jnp
from jax import lax
from jax.experimental import pallas as pl
from jax.experimental.pallas import tpu as pltpu

N_DEV = 4


def kernel(x, pi):
    _, m, n = x.shape

    def body(pi_ref, x_ref, out_ref, send_buf, send_sem, recv_sem):
        my = lax.axis_index("i")
        dst = pi_ref[my]

        send_buf[...] = x_ref[...].astype(jnp.bfloat16)
        rdma = pltpu.make_async_remote_copy(
            src_ref=send_buf,
            dst_ref=out_ref,
            send_sem=send_sem,
            recv_sem=recv_sem,
            device_id=(dst,),
            device_id_type=pl.DeviceIdType.MESH,
        )
        rdma.start()
        rdma.wait()

    return pl.pallas_call(
        body,
        out_shape=jax.ShapeDtypeStruct((1, m, n), jnp.bfloat16),
        in_specs=[
            pl.BlockSpec(memory_space=pltpu.SMEM),
            pl.BlockSpec(memory_space=pltpu.VMEM),
        ],
        out_specs=pl.BlockSpec(memory_space=pltpu.VMEM),
        scratch_shapes=[
            pltpu.VMEM((1, m, n), jnp.bfloat16),
            pltpu.SemaphoreType.DMA,
            pltpu.SemaphoreType.DMA,
        ],
    )(pi, x)


# device time: 59595 ns/iter; 1.0001x vs baseline; 1.0001x over previous
import jax
import jax.numpy as jnp
from jax import lax
from jax.experimental import pallas as pl
from jax.experimental.pallas import tpu as pltpu

N_DEV = 4
N_CHUNKS = 16


def kernel(x, pi):
    _, m, n = x.shape
    ch = m // N_CHUNKS

    def body(pi_ref, x_ref, out_ref, f32_buf, send_buf, load_sems,
             send_sems, recv_sems):
        my = lax.axis_index("i")
        dst = pi_ref[my]

        def load(k, slot):
            cp = pltpu.make_async_copy(
                x_ref.at[0, pl.ds(k * ch, ch), :],
                f32_buf.at[slot],
                load_sems.at[slot],
            )
            cp.start()
            return cp

        rdmas = []
        loads = [load(0, 0), None]
        for k in range(N_CHUNKS):
            slot = k % 2
            if k + 1 < N_CHUNKS:
                loads[(k + 1) % 2] = load(k + 1, (k + 1) % 2)
            loads[slot].wait()
            rows = pl.ds(k * ch, ch)
            send_buf[rows, :] = f32_buf[slot].astype(jnp.bfloat16)
            rdma = pltpu.make_async_remote_copy(
                src_ref=send_buf.at[rows, :],
                dst_ref=out_ref.at[0, rows, :],
                send_sem=send_sems.at[k],
                recv_sem=recv_sems.at[k],
                device_id=(dst,),
                device_id_type=pl.DeviceIdType.MESH,
            )
            rdma.start()
            rdmas.append(rdma)

        for rdma in rdmas:
            rdma.wait_send()
        for rdma in rdmas:
            rdma.wait_recv()

    return pl.pallas_call(
        body,
        out_shape=jax.ShapeDtypeStruct((1, m, n), jnp.bfloat16),
        in_specs=[
            pl.BlockSpec(memory_space=pltpu.SMEM),
            pl.BlockSpec(memory_space=pl.ANY),
        ],
        out_specs=pl.BlockSpec(memory_space=pl.ANY),
        scratch_shapes=[
            pltpu.VMEM((2, ch, n), jnp.float32),
            pltpu.VMEM((m, n), jnp.bfloat16),
            pltpu.SemaphoreType.DMA((2,)),
            pltpu.SemaphoreType.DMA((N_CHUNKS,)),
            pltpu.SemaphoreType.DMA((N_CHUNKS,)),
        ],
    )(pi, x)


# device time: 56096 ns/iter; 1.0625x vs baseline; 1.0624x over previous
import jax
import jax.numpy as jnp
from jax import lax
from jax.experimental import pallas as pl
from jax.experimental.pallas import tpu as pltpu

sem_signal = getattr(pl, "semaphore_signal", None) or pltpu.semaphore_signal
sem_wait = getattr(pl, "semaphore_wait", None) or pltpu.semaphore_wait

N_DEV = 4
N_CHUNKS = 8


def kernel(x, pi):
    _, m, n = x.shape
    ch = m // N_CHUNKS

    def body(pi_ref, x_ref, out_ref, f32_buf, send_buf, load_sems,
             send_sems, recv_sems):
        my = lax.axis_index("i")
        dst = pi_ref[my]
        src = jnp.int32(0)
        for j in range(N_DEV):
            src = jnp.where(pi_ref[j] == my, jnp.int32(j), src)

        barrier = pltpu.get_barrier_semaphore()
        sem_signal(barrier, inc=1, device_id=(dst,),
                   device_id_type=pl.DeviceIdType.MESH)
        sem_signal(barrier, inc=1, device_id=(src,),
                   device_id_type=pl.DeviceIdType.MESH)

        def load(k, slot):
            cp = pltpu.make_async_copy(
                x_ref.at[0, pl.ds(k * ch, ch), :],
                f32_buf.at[slot],
                load_sems.at[slot],
            )
            cp.start()
            return cp

        rdmas = []
        loads = [load(0, 0), None]
        for k in range(N_CHUNKS):
            slot = k % 2
            if k + 1 < N_CHUNKS:
                loads[(k + 1) % 2] = load(k + 1, (k + 1) % 2)
            loads[slot].wait()
            rows = pl.ds(k * ch, ch)
            send_buf[rows, :] = f32_buf[slot].astype(jnp.bfloat16)
            if k == 0:
                sem_wait(barrier, 2)
            rdma = pltpu.make_async_remote_copy(
                src_ref=send_buf.at[rows, :],
                dst_ref=out_ref.at[0, rows, :],
                send_sem=send_sems.at[k],
                recv_sem=recv_sems.at[k],
                device_id=(dst,),
                device_id_type=pl.DeviceIdType.MESH,
            )
            rdma.start()
            rdmas.append(rdma)

        for rdma in rdmas:
            rdma.wait_send()
        for rdma in rdmas:
            rdma.wait_recv()

    return pl.pallas_call(
        body,
        out_shape=jax.ShapeDtypeStruct((1, m, n), jnp.bfloat16),
        in_specs=[
            pl.BlockSpec(memory_space=pltpu.SMEM),
            pl.BlockSpec(memory_space=pl.ANY),
        ],
        out_specs=pl.BlockSpec(memory_space=pl.ANY),
        scratch_shapes=[
            pltpu.VMEM((2, ch, n), jnp.float32),
            pltpu.VMEM((m, n), jnp.bfloat16),
            pltpu.SemaphoreType.DMA((2,)),
            pltpu.SemaphoreType.DMA((N_CHUNKS,)),
            pltpu.SemaphoreType.DMA((N_CHUNKS,)),
        ],
        compiler_params=pltpu.CompilerParams(collective_id=0),
    )(pi, x)
